# baseline (speedup 1.0000x reference)
import functools

import jax
import jax.numpy as jnp
from jax import lax
from jax.experimental import pallas as pl
from jax.experimental.pallas import tpu as pltpu
from jax.experimental.pallas import tpu_sc as plsc

N = 10000
E = 320000
D_IN = 128
HID = 50
NUM_CLASS = 40
W1P = 56
W2P = 40

NC = 2
NS = 16
NW = NC * NS
EPT = E // NW
CHUNK = 128
NCHUNK = EPT // CHUNK
TAIL = EPT - NCHUNK * CHUNK
NBUF = 8
NP = 10240
RPT = NP // NS
ZROWS = 128
DEGW = 8

_mesh = plsc.VectorSubcoreMesh(
    core_axis_name="c", subcore_axis_name="s", num_cores=NC, num_subcores=NS
)
_sc_params = pltpu.CompilerParams(use_tc_tiling_on_sc=False)


@functools.partial(
    pl.kernel,
    out_type=jax.ShapeDtypeStruct((NC, NP, DEGW), jnp.float32),
    mesh=_mesh,
    compiler_params=_sc_params,
    scratch_types=[
        pltpu.VMEM((NCHUNK, CHUNK), jnp.int32),
        pltpu.VMEM((1, TAIL), jnp.int32),
        pltpu.VMEM((CHUNK, DEGW), jnp.float32),
        pltpu.VMEM((ZROWS, DEGW), jnp.float32),
        pltpu.VMEM_SHARED((NP, DEGW), jnp.float32),
        pltpu.SemaphoreType.DMA,
    ],
)
def _deg_kernel(dst_hbm, dstt_hbm, ones_hbm, zdeg_hbm, out_hbm, dstv, dsttv,
                onesv, zv, acc, sem):
    cid = lax.axis_index("c")
    sid = lax.axis_index("s")
    wid = cid * NS + sid

    pltpu.sync_copy(zdeg_hbm, zv)
    for r in range(RPT // ZROWS):
        pltpu.sync_copy(zv, acc.at[pl.ds(sid * RPT + r * ZROWS, ZROWS)])
    pltpu.sync_copy(ones_hbm, onesv)
    pltpu.sync_copy(dst_hbm.at[wid], dstv)
    pltpu.sync_copy(dstt_hbm.at[wid], dsttv)
    plsc.subcore_barrier()

    def fire(j, carry):
        pltpu.async_copy(onesv, acc.at[dstv.at[j]], sem, add=True)
        return carry

    def drain(j, carry):
        pltpu.make_async_copy(onesv, acc.at[dstv.at[j]], sem).wait()
        return carry

    lax.fori_loop(0, NCHUNK, fire, 0)
    pltpu.async_copy(onesv.at[pl.ds(0, TAIL)], acc.at[dsttv.at[0]], sem,
                     add=True)
    lax.fori_loop(0, NCHUNK, drain, 0)
    pltpu.make_async_copy(onesv.at[pl.ds(0, TAIL)], acc.at[dsttv.at[0]],
                          sem).wait()
    plsc.subcore_barrier()

    for r in range(RPT // ZROWS):
        row0 = sid * RPT + r * ZROWS
        pltpu.sync_copy(acc.at[pl.ds(row0, ZROWS)],
                        out_hbm.at[cid, pl.ds(row0, ZROWS)])


def _make_agg_kernel(width):
    @functools.partial(
        pl.kernel,
        out_type=jax.ShapeDtypeStruct((NC, NP, width), jnp.float32),
        mesh=_mesh,
        compiler_params=_sc_params,
        scratch_types=[
            pltpu.VMEM((NCHUNK, CHUNK), jnp.int32),
            pltpu.VMEM((NCHUNK, CHUNK), jnp.int32),
            pltpu.VMEM((1, TAIL), jnp.int32),
            pltpu.VMEM((1, TAIL), jnp.int32),
            [pltpu.VMEM((CHUNK, width), jnp.float32) for _ in range(NBUF)],
            pltpu.VMEM((ZROWS, width), jnp.float32),
            pltpu.VMEM_SHARED((NP, width), jnp.float32),
            [pltpu.SemaphoreType.DMA for _ in range(NBUF)],
            [pltpu.SemaphoreType.DMA for _ in range(NBUF)],
        ],
    )
    def agg(table_hbm, src_hbm, dst_hbm, srct_hbm, dstt_hbm, zpad_hbm,
            out_hbm, srcv, dstv, srctv, dsttv, bufs, zv, acc, gsems, ssems):
        cid = lax.axis_index("c")
        sid = lax.axis_index("s")
        wid = cid * NS + sid

        pltpu.sync_copy(zpad_hbm, zv)
        for r in range(RPT // ZROWS):
            pltpu.sync_copy(zv, acc.at[pl.ds(sid * RPT + r * ZROWS, ZROWS)])
        pltpu.sync_copy(src_hbm.at[wid], srcv)
        pltpu.sync_copy(dst_hbm.at[wid], dstv)
        pltpu.sync_copy(srct_hbm.at[wid], srctv)
        pltpu.sync_copy(dstt_hbm.at[wid], dsttv)
        plsc.subcore_barrier()

        def fire_g(j, k):
            pltpu.async_copy(table_hbm.at[srcv.at[j]], bufs[k], gsems[k])

        def drain_g(j, k):
            pltpu.make_async_copy(table_hbm.at[srcv.at[j]], bufs[k],
                                  gsems[k]).wait()

        def fire_s(j, k):
            pltpu.async_copy(bufs[k], acc.at[dstv.at[j]], ssems[k], add=True)

        def drain_s(j, k):
            pltpu.make_async_copy(bufs[k], acc.at[dstv.at[j]],
                                  ssems[k]).wait()

        for k in range(NBUF):
            fire_g(k, k)

        def group(i, carry):
            j0 = NBUF * i
            for k in range(NBUF):
                drain_g(j0 + k, k)
                fire_s(j0 + k, k)
            for k in range(NBUF):
                drain_s(j0 + k, k)
                fire_g(j0 + NBUF + k, k)
            return carry

        nsteady = NCHUNK // NBUF - 1
        lax.fori_loop(0, nsteady, group, 0)

        j0 = nsteady * NBUF
        rem = NCHUNK - j0 - NBUF
        for k in range(NBUF):
            drain_g(j0 + k, k)
            fire_s(j0 + k, k)
        for k in range(rem):
            drain_s(j0 + k, k)
            fire_g(j0 + NBUF + k, k)
        for k in range(rem, NBUF):
            drain_s(j0 + k, k)
        for k in range(rem):
            drain_g(j0 + NBUF + k, k)
            fire_s(j0 + NBUF + k, k)
        for k in range(rem):
            drain_s(j0 + NBUF + k, k)
        tbuf = bufs[NBUF - 1].at[pl.ds(0, TAIL)]
        pltpu.async_copy(table_hbm.at[srctv.at[0]], tbuf, gsems[0])
        pltpu.make_async_copy(table_hbm.at[srctv.at[0]], tbuf,
                              gsems[0]).wait()
        pltpu.sync_copy(tbuf, acc.at[dsttv.at[0]], add=True)
        plsc.subcore_barrier()

        for r in range(RPT // ZROWS):
            row0 = sid * RPT + r * ZROWS
            pltpu.sync_copy(acc.at[pl.ds(row0, ZROWS)],
                            out_hbm.at[cid, pl.ds(row0, ZROWS)])

    return agg


_agg1 = _make_agg_kernel(W1P)
_agg2 = _make_agg_kernel(W2P)


def _dinv(degp_ref):
    deg = degp_ref[0, :N, 0:1] + degp_ref[1, :N, 0:1] + 1.0
    return lax.rsqrt(deg)


def _t1_body(x_ref, w1_ref, degp_ref, h1p_ref):
    h = jnp.dot(x_ref[...], w1_ref[...], preferred_element_type=jnp.float32)
    h1p_ref[...] = h * _dinv(degp_ref)


def _t2_body(aggp_ref, h1p_ref, degp_ref, b1_ref, w2_ref, h2p_ref):
    dinv = _dinv(degp_ref)
    agg = aggp_ref[0, :N] + aggp_ref[1, :N]
    z = jax.nn.relu(dinv * (agg + h1p_ref[...]) + b1_ref[...])
    h2p_ref[...] = jnp.dot(z, w2_ref[...],
                           preferred_element_type=jnp.float32) * dinv


def _t3_body(aggp_ref, h2p_ref, degp_ref, b2_ref, out_ref):
    dinv = _dinv(degp_ref)
    agg = aggp_ref[0, :N] + aggp_ref[1, :N]
    logits = dinv * (agg + h2p_ref[...]) + b2_ref[...]
    m = jnp.max(logits, axis=1, keepdims=True)
    lse = jnp.log(jnp.sum(jnp.exp(logits - m), axis=1, keepdims=True))
    out_ref[...] = logits - m - lse


_t1 = pl.pallas_call(
    _t1_body, out_shape=jax.ShapeDtypeStruct((N, W1P), jnp.float32))
_t2 = pl.pallas_call(
    _t2_body, out_shape=jax.ShapeDtypeStruct((N, W2P), jnp.float32))
_t3 = pl.pallas_call(
    _t3_body, out_shape=jax.ShapeDtypeStruct((N, W2P), jnp.float32))


def kernel(x, edge_index, W1, b1, W2, b2):
    w1p = jnp.zeros((D_IN, W1P), jnp.float32).at[:, :HID].set(W1)
    w2p = jnp.zeros((W1P, W2P), jnp.float32).at[:HID, :].set(W2)
    b1p = jnp.zeros((1, W1P), jnp.float32).at[0, :HID].set(b1)
    b2p = b2.reshape(1, W2P)
    e0 = edge_index[0].reshape(NW, EPT)
    e1 = edge_index[1].reshape(NW, EPT)
    src = e0[:, :NCHUNK * CHUNK].reshape(NW, NCHUNK, CHUNK)
    dst = e1[:, :NCHUNK * CHUNK].reshape(NW, NCHUNK, CHUNK)
    srct = e0[:, NCHUNK * CHUNK:].reshape(NW, 1, TAIL)
    dstt = e1[:, NCHUNK * CHUNK:].reshape(NW, 1, TAIL)
    ones_deg = jnp.ones((CHUNK, DEGW), jnp.float32)
    zdeg = jnp.zeros((ZROWS, DEGW), jnp.float32)
    zpad1 = jnp.zeros((ZROWS, W1P), jnp.float32)
    zpad2 = jnp.zeros((ZROWS, W2P), jnp.float32)

    degp = _deg_kernel(dst, dstt, ones_deg, zdeg)
    h1p = _t1(x, w1p, degp)
    agg1 = _agg1(h1p, src, dst, srct, dstt, zpad1)
    h2p = _t2(agg1, h1p, degp, b1p, w2p)
    agg2 = _agg2(h2p, src, dst, srct, dstt, zpad2)
    return _t3(agg2, h2p, degp, b2p)


# speedup vs baseline: 1.0155x; 1.0155x over previous
import functools

import jax
import jax.numpy as jnp
from jax import lax
from jax.experimental import pallas as pl
from jax.experimental.pallas import tpu as pltpu
from jax.experimental.pallas import tpu_sc as plsc

N = 10000
E = 320000
D_IN = 128
HID = 50
NUM_CLASS = 40
W1P = 56
W2P = 40

NC = 2
NS = 16
NW = NC * NS
EPT = E // NW
CHUNK = 80
NCHUNK = EPT // CHUNK
NBUF = 8
NP = 10240
RPT = NP // NS
ZROWS = 128
DEGW = 8

_mesh = plsc.VectorSubcoreMesh(
    core_axis_name="c", subcore_axis_name="s", num_cores=NC, num_subcores=NS
)
_sc_params = pltpu.CompilerParams(use_tc_tiling_on_sc=False)


@functools.partial(
    pl.kernel,
    out_type=jax.ShapeDtypeStruct((NC, NP, DEGW), jnp.float32),
    mesh=_mesh,
    compiler_params=_sc_params,
    scratch_types=[
        pltpu.VMEM((NCHUNK, CHUNK), jnp.int32),
        pltpu.VMEM((CHUNK, DEGW), jnp.float32),
        pltpu.VMEM((ZROWS, DEGW), jnp.float32),
        pltpu.VMEM_SHARED((NP, DEGW), jnp.float32),
        pltpu.SemaphoreType.DMA,
    ],
)
def _deg_kernel(dst_hbm, ones_hbm, zdeg_hbm, out_hbm, dstv, onesv, zv, acc,
                sem):
    cid = lax.axis_index("c")
    sid = lax.axis_index("s")
    wid = cid * NS + sid

    pltpu.sync_copy(zdeg_hbm, zv)
    for r in range(RPT // ZROWS):
        pltpu.sync_copy(zv, acc.at[pl.ds(sid * RPT + r * ZROWS, ZROWS)])
    pltpu.sync_copy(ones_hbm, onesv)
    pltpu.sync_copy(dst_hbm.at[wid], dstv)
    plsc.subcore_barrier()

    def fire(j, carry):
        pltpu.async_copy(onesv, acc.at[dstv.at[j]], sem, add=True)
        return carry

    def drain(j, carry):
        pltpu.make_async_copy(onesv, acc.at[dstv.at[j]], sem).wait()
        return carry

    lax.fori_loop(0, NCHUNK, fire, 0)
    lax.fori_loop(0, NCHUNK, drain, 0)
    plsc.subcore_barrier()

    for r in range(RPT // ZROWS):
        row0 = sid * RPT + r * ZROWS
        pltpu.sync_copy(acc.at[pl.ds(row0, ZROWS)],
                        out_hbm.at[cid, pl.ds(row0, ZROWS)])


def _make_agg_kernel(width):
    @functools.partial(
        pl.kernel,
        out_type=jax.ShapeDtypeStruct((NC, NP, width), jnp.float32),
        mesh=_mesh,
        compiler_params=_sc_params,
        scratch_types=[
            pltpu.VMEM((NCHUNK, CHUNK), jnp.int32),
            pltpu.VMEM((NCHUNK, CHUNK), jnp.int32),
            [pltpu.VMEM((CHUNK, width), jnp.float32) for _ in range(NBUF)],
            pltpu.VMEM((ZROWS, width), jnp.float32),
            pltpu.VMEM_SHARED((NP, width), jnp.float32),
            [pltpu.SemaphoreType.DMA for _ in range(NBUF)],
            [pltpu.SemaphoreType.DMA for _ in range(NBUF)],
        ],
    )
    def agg(table_hbm, src_hbm, dst_hbm, zpad_hbm, out_hbm,
            srcv, dstv, bufs, zv, acc, gsems, ssems):
        cid = lax.axis_index("c")
        sid = lax.axis_index("s")
        wid = cid * NS + sid

        pltpu.sync_copy(zpad_hbm, zv)
        for r in range(RPT // ZROWS):
            pltpu.sync_copy(zv, acc.at[pl.ds(sid * RPT + r * ZROWS, ZROWS)])
        pltpu.sync_copy(src_hbm.at[wid], srcv)
        pltpu.sync_copy(dst_hbm.at[wid], dstv)
        plsc.subcore_barrier()

        def fire_g(j, k):
            pltpu.async_copy(table_hbm.at[srcv.at[j]], bufs[k], gsems[k])

        def drain_g(j, k):
            pltpu.make_async_copy(table_hbm.at[srcv.at[j]], bufs[k],
                                  gsems[k]).wait()

        def fire_s(j, k):
            pltpu.async_copy(bufs[k], acc.at[dstv.at[j]], ssems[k], add=True)

        def drain_s(j, k):
            pltpu.make_async_copy(bufs[k], acc.at[dstv.at[j]],
                                  ssems[k]).wait()

        for k in range(NBUF):
            fire_g(k, k)

        def group(i, carry):
            j0 = NBUF * i
            for k in range(NBUF):
                drain_g(j0 + k, k)
                fire_s(j0 + k, k)
            for k in range(NBUF):
                drain_s(j0 + k, k)
                fire_g(j0 + NBUF + k, k)
            return carry

        nsteady = NCHUNK // NBUF - 1
        lax.fori_loop(0, nsteady, group, 0)

        j0 = nsteady * NBUF
        rem = NCHUNK - j0 - NBUF
        for k in range(NBUF):
            drain_g(j0 + k, k)
            fire_s(j0 + k, k)
        for k in range(rem):
            drain_s(j0 + k, k)
            fire_g(j0 + NBUF + k, k)
        for k in range(rem, NBUF):
            drain_s(j0 + k, k)
        for k in range(rem):
            drain_g(j0 + NBUF + k, k)
            fire_s(j0 + NBUF + k, k)
        for k in range(rem):
            drain_s(j0 + NBUF + k, k)
        plsc.subcore_barrier()

        for r in range(RPT // ZROWS):
            row0 = sid * RPT + r * ZROWS
            pltpu.sync_copy(acc.at[pl.ds(row0, ZROWS)],
                            out_hbm.at[cid, pl.ds(row0, ZROWS)])

    return agg


_agg1 = _make_agg_kernel(W1P)
_agg2 = _make_agg_kernel(W2P)


def _dinv(degp_ref):
    deg = degp_ref[0, :N, 0:1] + degp_ref[1, :N, 0:1] + 1.0
    return lax.rsqrt(deg)


def _t1_body(x_ref, w1_ref, degp_ref, h1p_ref):
    h = jnp.dot(x_ref[...], w1_ref[...], preferred_element_type=jnp.float32)
    h1p_ref[...] = h * _dinv(degp_ref)


def _t2_body(aggp_ref, h1p_ref, degp_ref, b1_ref, w2_ref, h2p_ref):
    dinv = _dinv(degp_ref)
    agg = aggp_ref[0, :N] + aggp_ref[1, :N]
    z = jax.nn.relu(dinv * (agg + h1p_ref[...]) + b1_ref[...])
    h2p_ref[...] = jnp.dot(z, w2_ref[...],
                           preferred_element_type=jnp.float32) * dinv


def _t3_body(aggp_ref, h2p_ref, degp_ref, b2_ref, out_ref):
    dinv = _dinv(degp_ref)
    agg = aggp_ref[0, :N] + aggp_ref[1, :N]
    logits = dinv * (agg + h2p_ref[...]) + b2_ref[...]
    m = jnp.max(logits, axis=1, keepdims=True)
    lse = jnp.log(jnp.sum(jnp.exp(logits - m), axis=1, keepdims=True))
    out_ref[...] = logits - m - lse


_t1 = pl.pallas_call(
    _t1_body, out_shape=jax.ShapeDtypeStruct((N, W1P), jnp.float32))
_t2 = pl.pallas_call(
    _t2_body, out_shape=jax.ShapeDtypeStruct((N, W2P), jnp.float32))
_t3 = pl.pallas_call(
    _t3_body, out_shape=jax.ShapeDtypeStruct((N, W2P), jnp.float32))


def kernel(x, edge_index, W1, b1, W2, b2):
    w1p = jnp.zeros((D_IN, W1P), jnp.float32).at[:, :HID].set(W1)
    w2p = jnp.zeros((W1P, W2P), jnp.float32).at[:HID, :].set(W2)
    b1p = jnp.zeros((1, W1P), jnp.float32).at[0, :HID].set(b1)
    b2p = b2.reshape(1, W2P)
    src = edge_index[0].reshape(NW, NCHUNK, CHUNK)
    dst = edge_index[1].reshape(NW, NCHUNK, CHUNK)
    ones_deg = jnp.ones((CHUNK, DEGW), jnp.float32)
    zdeg = jnp.zeros((ZROWS, DEGW), jnp.float32)
    zpad1 = jnp.zeros((ZROWS, W1P), jnp.float32)
    zpad2 = jnp.zeros((ZROWS, W2P), jnp.float32)

    degp = _deg_kernel(dst, ones_deg, zdeg)
    h1p = _t1(x, w1p, degp)
    agg1 = _agg1(h1p, src, dst, zpad1)
    h2p = _t2(agg1, h1p, degp, b1p, w2p)
    agg2 = _agg2(h2p, src, dst, zpad2)
    return _t3(agg2, h2p, degp, b2p)
